# baseline (speedup 1.0000x reference)
import functools

import jax
import jax.numpy as jnp
from jax import lax
from jax.experimental import pallas as pl
from jax.experimental.pallas import tpu as pltpu
from jax.experimental.pallas import tpu_sc as plsc

N = 10000
E = 320000
F = 128
OUT = 64
NP = 10240
NW = 32
EW = E // NW
CE = 40
NCHUNK = EW // CE
assert NCHUNK == 250
RPT = NP // 16

_mesh = plsc.VectorSubcoreMesh(core_axis_name="c", subcore_axis_name="s")


@functools.partial(
    pl.kernel,
    out_type=jax.ShapeDtypeStruct((NW * NP,), jnp.float32),
    mesh=_mesh,
    scratch_types=[
        pltpu.VMEM((EW,), jnp.int32),
        pltpu.VMEM((NP,), jnp.float32),
    ],
    compiler_params=pltpu.CompilerParams(needs_layout_passes=False),
)
def _deg_kernel(dst_hbm, out_hbm, idx_v, hist_v):
    c = lax.axis_index("c")
    s = lax.axis_index("s")
    w = s * 2 + c
    pltpu.sync_copy(dst_hbm.at[pl.ds(w * EW, EW)], idx_v)

    zeros16 = jnp.zeros((16,), jnp.float32)

    def zero_body(j, carry):
        hist_v[pl.ds(j * 16, 16)] = zeros16
        return carry

    lax.fori_loop(0, NP // 16, zero_body, 0)

    ones16 = jnp.ones((16,), jnp.float32)

    def add_body(j, carry):
        idx = idx_v[pl.ds(j * 16, 16)]
        plsc.addupdate_scatter(hist_v, [idx], ones16)
        return carry

    lax.fori_loop(0, EW // 16, add_body, 0)
    pltpu.sync_copy(hist_v, out_hbm.at[pl.ds(w * NP, NP)])


@functools.partial(
    pl.kernel,
    out_type=jax.ShapeDtypeStruct((2, NP, F), jnp.float32),
    mesh=_mesh,
    scratch_types=[
        pltpu.VMEM((EW,), jnp.int32),
        pltpu.VMEM((EW,), jnp.int32),
        pltpu.VMEM((4, CE, F), jnp.float32),
        pltpu.VMEM_SHARED((NP, F), jnp.float32),
        pltpu.SemaphoreType.DMA,
        pltpu.SemaphoreType.DMA,
        pltpu.SemaphoreType.DMA,
        pltpu.SemaphoreType.DMA,
        pltpu.SemaphoreType.DMA,
        pltpu.SemaphoreType.DMA,
        pltpu.SemaphoreType.DMA,
        pltpu.SemaphoreType.DMA,
    ],
)
def _agg_kernel(u_hbm, src_hbm, dst_hbm, zeros_hbm, out_hbm,
                sidx_v, didx_v, rows_v, acc,
                sg0, sg1, sg2, sg3, sa0, sa1, sa2, sa3):
    c = lax.axis_index("c")
    s = lax.axis_index("s")
    w = s * 2 + c
    r0 = s * RPT
    pltpu.sync_copy(zeros_hbm.at[pl.ds(r0, RPT)], acc.at[pl.ds(r0, RPT)])
    pltpu.sync_copy(src_hbm.at[pl.ds(w * EW, EW)], sidx_v)
    pltpu.sync_copy(dst_hbm.at[pl.ds(w * EW, EW)], didx_v)
    plsc.subcore_barrier()

    bufs = [rows_v.at[0], rows_v.at[1], rows_v.at[2], rows_v.at[3]]
    sgs = [sg0, sg1, sg2, sg3]
    sas = [sa0, sa1, sa2, sa3]

    def gather(k, b):
        pltpu.async_copy(u_hbm.at[sidx_v.at[pl.ds(k * CE, CE)]], bufs[b], sgs[b])

    def addi(k, b):
        pltpu.async_copy(bufs[b], acc.at[didx_v.at[pl.ds(k * CE, CE)]],
                         sas[b], add=True)

    def wait_g(b):
        pltpu.make_async_copy(u_hbm.at[pl.ds(0, CE)], bufs[b], sgs[b]).wait()

    def wait_a(b):
        pltpu.make_async_copy(u_hbm.at[pl.ds(0, CE)], bufs[b], sas[b]).wait()

    gather(0, 0)
    gather(1, 1)
    wait_g(0); addi(0, 0); gather(2, 2)
    wait_g(1); addi(1, 1); gather(3, 3)
    wait_g(2); addi(2, 2); wait_a(0); gather(4, 0)
    wait_g(3); addi(3, 3); wait_a(1); gather(5, 1)

    def body(kk, carry):
        k0 = 4 * kk
        wait_g(0); addi(k0, 0);     wait_a(2); gather(k0 + 2, 2)
        wait_g(1); addi(k0 + 1, 1); wait_a(3); gather(k0 + 3, 3)
        wait_g(2); addi(k0 + 2, 2); wait_a(0); gather(k0 + 4, 0)
        wait_g(3); addi(k0 + 3, 3); wait_a(1); gather(k0 + 5, 1)
        return carry

    lax.fori_loop(1, 62, body, 0)
    wait_g(0); addi(248, 0)
    wait_g(1); addi(249, 1)
    wait_a(2); wait_a(3); wait_a(0); wait_a(1)

    plsc.subcore_barrier()
    pltpu.sync_copy(acc.at[pl.ds(r0, RPT)], out_hbm.at[c, pl.ds(r0, RPT)])


def _prep_body(hist_ref, x_ref, dinv_ref, u0_ref):
    deg = jnp.sum(hist_ref[...], axis=0) + 1.0
    dinv = lax.rsqrt(deg)
    dinv_ref[...] = dinv
    u0_ref[...] = x_ref[...] * dinv[:, None]


_prep = pl.pallas_call(
    _prep_body,
    out_shape=(
        jax.ShapeDtypeStruct((NP,), jnp.float32),
        jax.ShapeDtypeStruct((NP, F), jnp.float32),
    ),
)


BM = 1024


def _layer1_body(parts_ref, u0_ref, dinv_ref, w1_ref, b1_ref, u1_ref):
    su = parts_ref[0] + parts_ref[1] + u0_ref[...]
    dinv = dinv_ref[...]
    ax = su * dinv[:, None]
    h = jnp.dot(ax, w1_ref[...], preferred_element_type=jnp.float32)
    h = jnp.maximum(h + b1_ref[...][None, :], 0.0)
    u1_ref[...] = h * dinv[:, None]


_layer1 = pl.pallas_call(
    _layer1_body,
    grid=(NP // BM,),
    in_specs=[
        pl.BlockSpec((2, BM, F), lambda i: (0, i, 0)),
        pl.BlockSpec((BM, F), lambda i: (i, 0)),
        pl.BlockSpec((BM,), lambda i: (i,)),
        pl.BlockSpec((F, F), lambda i: (0, 0)),
        pl.BlockSpec((F,), lambda i: (0,)),
    ],
    out_specs=pl.BlockSpec((BM, F), lambda i: (i, 0)),
    out_shape=jax.ShapeDtypeStruct((NP, F), jnp.float32),
)


def _layer23_body(parts_ref, u1_ref, dinv_ref, w2_ref, b2_ref, o2_ref):
    su = parts_ref[0] + parts_ref[1] + u1_ref[...]
    ah = su * dinv_ref[...][:, None]
    o = jnp.dot(ah, w2_ref[...], preferred_element_type=jnp.float32)
    o2_ref[...] = o + b2_ref[...][None, :]


_layer23 = pl.pallas_call(
    _layer23_body,
    grid=(NP // BM,),
    in_specs=[
        pl.BlockSpec((2, BM, F), lambda i: (0, i, 0)),
        pl.BlockSpec((BM, F), lambda i: (i, 0)),
        pl.BlockSpec((BM,), lambda i: (i,)),
        pl.BlockSpec((F, 2 * OUT), lambda i: (0, 0)),
        pl.BlockSpec((2 * OUT,), lambda i: (0,)),
    ],
    out_specs=pl.BlockSpec((BM, 2 * OUT), lambda i: (i, 0)),
    out_shape=jax.ShapeDtypeStruct((NP, 2 * OUT), jnp.float32),
)


def kernel(x, edge_index, W1, b1, Wmu, bmu, Wls, bls):
    src = edge_index[0]
    dst = edge_index[1]
    xp = jnp.zeros((NP, F), jnp.float32).at[:N].set(x)
    zeros = jnp.zeros((NP, F), jnp.float32)

    hist = _deg_kernel(dst).reshape(NW, NP)
    dinv, u0 = _prep(hist, xp)

    parts1 = _agg_kernel(u0, src, dst, zeros)
    u1 = _layer1(parts1, u0, dinv, W1, b1)

    parts2 = _agg_kernel(u1, src, dst, zeros)
    W2 = jnp.concatenate([Wmu, Wls], axis=1)
    b2 = jnp.concatenate([bmu, bls])
    o2 = _layer23(parts2, u1, dinv, W2, b2)

    return (o2[:N, :OUT], o2[:N, OUT:])


# speedup vs baseline: 1.2673x; 1.2673x over previous
import functools

import jax
import jax.numpy as jnp
from jax import lax
from jax.experimental import pallas as pl
from jax.experimental.pallas import tpu as pltpu
from jax.experimental.pallas import tpu_sc as plsc

N = 10000
E = 320000
F = 128
OUT = 64
NP = 10240
NW = 32
EW = E // NW
CE = 80
NCHUNK = EW // CE
assert NCHUNK == 125
RPT = NP // 16

_mesh = plsc.VectorSubcoreMesh(core_axis_name="c", subcore_axis_name="s")


@functools.partial(
    pl.kernel,
    out_type=jax.ShapeDtypeStruct((NW * NP,), jnp.float32),
    mesh=_mesh,
    scratch_types=[
        pltpu.VMEM((EW,), jnp.int32),
        pltpu.VMEM((NP,), jnp.float32),
    ],
    compiler_params=pltpu.CompilerParams(needs_layout_passes=False),
)
def _deg_kernel(dst_hbm, out_hbm, idx_v, hist_v):
    c = lax.axis_index("c")
    s = lax.axis_index("s")
    w = s * 2 + c
    pltpu.sync_copy(dst_hbm.at[pl.ds(w * EW, EW)], idx_v)

    zeros16 = jnp.zeros((16,), jnp.float32)

    def zero_body(j, carry):
        hist_v[pl.ds(j * 16, 16)] = zeros16
        return carry

    lax.fori_loop(0, NP // 16, zero_body, 0)

    ones16 = jnp.ones((16,), jnp.float32)

    def add_body(j, carry):
        idx = idx_v[pl.ds(j * 16, 16)]
        plsc.addupdate_scatter(hist_v, [idx], ones16)
        return carry

    lax.fori_loop(0, EW // 16, add_body, 0)
    pltpu.sync_copy(hist_v, out_hbm.at[pl.ds(w * NP, NP)])


@functools.partial(
    pl.kernel,
    out_type=jax.ShapeDtypeStruct((2, NP, F), jnp.float32),
    mesh=_mesh,
    scratch_types=[
        pltpu.VMEM((EW,), jnp.int32),
        pltpu.VMEM((3, CE), jnp.int32),
        pltpu.VMEM((3, CE, F), jnp.float32),
        pltpu.VMEM_SHARED((NP, F), jnp.float32),
        pltpu.SemaphoreType.DMA,
        pltpu.SemaphoreType.DMA,
        pltpu.SemaphoreType.DMA,
        pltpu.SemaphoreType.DMA,
        pltpu.SemaphoreType.DMA,
        pltpu.SemaphoreType.DMA,
        pltpu.SemaphoreType.DMA,
        pltpu.SemaphoreType.DMA,
        pltpu.SemaphoreType.DMA,
    ],
)
def _agg_kernel(u_hbm, src_hbm, dst_hbm, zeros_hbm, out_hbm,
                sidx_v, didx_v, rows_v, acc,
                sg0, sg1, sg2, sa0, sa1, sa2, si0, si1, si2):
    c = lax.axis_index("c")
    s = lax.axis_index("s")
    w = s * 2 + c
    r0 = s * RPT
    pltpu.sync_copy(zeros_hbm.at[pl.ds(r0, RPT)], acc.at[pl.ds(r0, RPT)])
    pltpu.sync_copy(src_hbm.at[pl.ds(w * EW, EW)], sidx_v)
    plsc.subcore_barrier()

    bufs = [rows_v.at[0], rows_v.at[1], rows_v.at[2]]
    ibufs = [didx_v.at[0], didx_v.at[1], didx_v.at[2]]
    sgs = [sg0, sg1, sg2]
    sas = [sa0, sa1, sa2]
    sis = [si0, si1, si2]

    def iload(k, b):
        pltpu.async_copy(dst_hbm.at[pl.ds(w * EW + k * CE, CE)], ibufs[b], sis[b])

    def gather(k, b):
        pltpu.async_copy(u_hbm.at[sidx_v.at[pl.ds(k * CE, CE)]], bufs[b], sgs[b])

    def addi(b):
        pltpu.async_copy(bufs[b], acc.at[ibufs[b]], sas[b], add=True)

    def wait_g(b):
        pltpu.make_async_copy(u_hbm.at[pl.ds(0, CE)], bufs[b], sgs[b]).wait()

    def wait_a(b):
        pltpu.make_async_copy(u_hbm.at[pl.ds(0, CE)], bufs[b], sas[b]).wait()

    def wait_i(b):
        pltpu.make_async_copy(dst_hbm.at[pl.ds(0, CE)], ibufs[b], sis[b]).wait()

    iload(0, 0); iload(1, 1)
    gather(0, 0); gather(1, 1)
    wait_g(0); wait_i(0); addi(0);            iload(2, 2); gather(2, 2)
    wait_g(1); wait_i(1); addi(1); wait_a(0); iload(3, 0); gather(3, 0)
    wait_g(2); wait_i(2); addi(2); wait_a(1); iload(4, 1); gather(4, 1)

    def body(kk, carry):
        j0 = 3 * kk
        wait_g(0); wait_i(0); addi(0); wait_a(2); iload(j0 + 2, 2); gather(j0 + 2, 2)
        wait_g(1); wait_i(1); addi(1); wait_a(0); iload(j0 + 3, 0); gather(j0 + 3, 0)
        wait_g(2); wait_i(2); addi(2); wait_a(1); iload(j0 + 4, 1); gather(j0 + 4, 1)
        return carry

    lax.fori_loop(1, 41, body, 0)
    wait_g(0); wait_i(0); addi(0); wait_a(2)
    wait_g(1); wait_i(1); addi(1); wait_a(0)
    wait_a(1)

    plsc.subcore_barrier()
    pltpu.sync_copy(acc.at[pl.ds(r0, RPT)], out_hbm.at[c, pl.ds(r0, RPT)])


def _prep_body(hist_ref, x_ref, dinv_ref, u0_ref):
    deg = jnp.sum(hist_ref[...], axis=0) + 1.0
    dinv = lax.rsqrt(deg)
    dinv_ref[...] = dinv
    u0_ref[...] = x_ref[...] * dinv[:, None]


_prep = pl.pallas_call(
    _prep_body,
    out_shape=(
        jax.ShapeDtypeStruct((NP,), jnp.float32),
        jax.ShapeDtypeStruct((NP, F), jnp.float32),
    ),
)


BM = 1024


def _layer1_body(parts_ref, u0_ref, dinv_ref, w1_ref, b1_ref, u1_ref):
    su = parts_ref[0] + parts_ref[1] + u0_ref[...]
    dinv = dinv_ref[...]
    ax = su * dinv[:, None]
    h = jnp.dot(ax, w1_ref[...], preferred_element_type=jnp.float32)
    h = jnp.maximum(h + b1_ref[...][None, :], 0.0)
    u1_ref[...] = h * dinv[:, None]


_layer1 = pl.pallas_call(
    _layer1_body,
    grid=(NP // BM,),
    in_specs=[
        pl.BlockSpec((2, BM, F), lambda i: (0, i, 0)),
        pl.BlockSpec((BM, F), lambda i: (i, 0)),
        pl.BlockSpec((BM,), lambda i: (i,)),
        pl.BlockSpec((F, F), lambda i: (0, 0)),
        pl.BlockSpec((F,), lambda i: (0,)),
    ],
    out_specs=pl.BlockSpec((BM, F), lambda i: (i, 0)),
    out_shape=jax.ShapeDtypeStruct((NP, F), jnp.float32),
)


def _layer23_body(parts_ref, u1_ref, dinv_ref, w2_ref, b2_ref, o2_ref):
    su = parts_ref[0] + parts_ref[1] + u1_ref[...]
    ah = su * dinv_ref[...][:, None]
    o = jnp.dot(ah, w2_ref[...], preferred_element_type=jnp.float32)
    o2_ref[...] = o + b2_ref[...][None, :]


_layer23 = pl.pallas_call(
    _layer23_body,
    grid=(NP // BM,),
    in_specs=[
        pl.BlockSpec((2, BM, F), lambda i: (0, i, 0)),
        pl.BlockSpec((BM, F), lambda i: (i, 0)),
        pl.BlockSpec((BM,), lambda i: (i,)),
        pl.BlockSpec((F, 2 * OUT), lambda i: (0, 0)),
        pl.BlockSpec((2 * OUT,), lambda i: (0,)),
    ],
    out_specs=pl.BlockSpec((BM, 2 * OUT), lambda i: (i, 0)),
    out_shape=jax.ShapeDtypeStruct((NP, 2 * OUT), jnp.float32),
)


def kernel(x, edge_index, W1, b1, Wmu, bmu, Wls, bls):
    src = edge_index[0]
    dst = edge_index[1]
    xp = jnp.zeros((NP, F), jnp.float32).at[:N].set(x)
    zeros = jnp.zeros((NP, F), jnp.float32)

    hist = _deg_kernel(dst).reshape(NW, NP)
    dinv, u0 = _prep(hist, xp)

    parts1 = _agg_kernel(u0, src, dst, zeros)
    u1 = _layer1(parts1, u0, dinv, W1, b1)

    parts2 = _agg_kernel(u1, src, dst, zeros)
    W2 = jnp.concatenate([Wmu, Wls], axis=1)
    b2 = jnp.concatenate([bmu, bls])
    o2 = _layer23(parts2, u1, dinv, W2, b2)

    return (o2[:N, :OUT], o2[:N, OUT:])
